# baseline (speedup 1.0000x reference)
import jax
import jax.numpy as jnp
import numpy as np
from jax import lax
from jax.experimental import pallas as pl
from jax.experimental.pallas import tpu as pltpu
from jax.experimental.pallas import tpu_sc as plsc

B = 1024
N = 20
D = 3
NF = 16
N_RBF = 16
N_LAYERS = 3
NP = N * N
NW = 32
GPW = B // NW

_F32 = jnp.float32
_I32 = jnp.int32


def _zero():
    return jnp.full((NF,), 0.0, dtype=_F32)


def _tree_sum(terms):
    while len(terms) > 1:
        terms = [terms[i] + terms[i + 1] for i in range(0, len(terms) - 1, 2)] \
            + ([terms[-1]] if len(terms) % 2 else [])
    return terms[0]


def _bcast_i32(s):
    return jnp.full((NF,), 0, dtype=_I32) + s


def _row(ref, r):
    return ref[pl.ds(r * NF, NF)]


def _set_row(ref, r, val):
    ref[pl.ds(r * NF, NF)] = val


def _sc_body(x_hbm, wr_hbm, wm_hbm, ws_hbm, idx_hbm, out_hbm,
             x_v, out_v, r_v, dh_v, rbf_v, f0_v, f1_v, a0_v, a1_v,
             wr_v, wm_v, ws_v, idx_v):
    wid = lax.axis_index("s") * 2 + lax.axis_index("c")
    base = wid * GPW
    pltpu.sync_copy(x_hbm.at[pl.ds(base * 8 * N, GPW * 8 * N)], x_v)
    pltpu.sync_copy(wr_hbm, wr_v)
    pltpu.sync_copy(wm_hbm, wm_v)
    pltpu.sync_copy(ws_hbm, ws_v)
    pltpu.sync_copy(idx_hbm, idx_v)
    wemb = _row(ws_v, 0)
    wvin = _row(ws_v, 1)
    wout = _row(ws_v, 2)
    centers = _row(ws_v, 3)
    eye = [_row(ws_v, 4 + k) for k in range(D)]

    def graph_body(g, _):
        def geom_body(q, _):
            sl = pl.ds(q * 16, 16)
            ii = idx_v[sl]
            jj = idx_v[pl.ds(NP + q * 16, 16)]
            d = []
            for k in range(D):
                cbase = _bcast_i32((g * 8 + k) * N)
                xi = plsc.load_gather(x_v, [cbase + ii])
                xj = plsc.load_gather(x_v, [cbase + jj])
                d.append(xj - xi)
            r2 = d[0] * d[0] + d[1] * d[1] + d[2] * d[2] + 1e-8
            yi = plsc.bitcast(r2, _I32)
            yi = jnp.full((NF,), 0x5F3759DF, dtype=_I32) \
                - lax.shift_right_logical(yi, 1)
            y = plsc.bitcast(yi, _F32)
            for _ in range(3):
                y = y * (1.5 - 0.5 * r2 * y * y)
            r = r2 * y
            r_v[sl] = jnp.where(ii == jj, jnp.full((NF,), 1e9, _F32), r)
            for k in range(D):
                dh_v[pl.ds(k * NP + q * 16, 16)] = d[k] * y
            return 0

        lax.fori_loop(0, NP // 16, geom_body, 0)

        def rbf_body(p, _):
            rb = plsc.load_gather(r_v, [_bcast_i32(p)])
            t = rb - centers
            _set_row(rbf_v, p, jnp.exp(-2.0 * t * t))
            return 0

        lax.fori_loop(0, NP, rbf_body, 0)

        def init_body(n, _):
            zf = plsc.load_gather(x_v, [_bcast_i32((g * 8 + 6) * N + n)])
            _set_row(f0_v, n, zf * wemb)
            for k in range(D):
                vk = plsc.load_gather(
                    x_v, [_bcast_i32((g * 8 + 3 + k) * N + n)])
                _set_row(f1_v, k * N + n, vk * wvin)
            return 0

        lax.fori_loop(0, N, init_body, 0)

        for l in range(N_LAYERS):
            w0rows = [_row(wr_v, (3 * l + 0) * NF + k) for k in range(N_RBF)]
            w1rows = [_row(wr_v, (3 * l + 1) * NF + k) for k in range(N_RBF)]
            w2rows = [_row(wr_v, (3 * l + 2) * NF + k) for k in range(N_RBF)]

            def agg_j(j, _):
                def agg_i(i, acc):
                    a0, ax, ay, az = acc
                    p = i * N + j
                    rv = _row(rbf_v, p)
                    w0 = _tree_sum([rv[k] * w0rows[k] for k in range(N_RBF)])
                    w1 = _tree_sum([rv[k] * w1rows[k] for k in range(N_RBF)])
                    w2 = _tree_sum([rv[k] * w2rows[k] for k in range(N_RBF)])
                    pb = _bcast_i32(p)
                    dhx = plsc.load_gather(dh_v, [pb])
                    dhy = plsc.load_gather(dh_v, [pb + NP])
                    dhz = plsc.load_gather(dh_v, [pb + 2 * NP])
                    a0 = a0 + w0 * _row(f0_v, i)
                    ax = ax + w1 * _row(f1_v, i) + dhx * w2
                    ay = ay + w1 * _row(f1_v, N + i) + dhy * w2
                    az = az + w1 * _row(f1_v, 2 * N + i) + dhz * w2
                    return a0, ax, ay, az

                a0, ax, ay, az = lax.fori_loop(
                    0, N, agg_i, (_zero(), _zero(), _zero(), _zero()))
                _set_row(a0_v, j, a0)
                _set_row(a1_v, j, ax)
                _set_row(a1_v, N + j, ay)
                _set_row(a1_v, 2 * N + j, az)
                return 0

            lax.fori_loop(0, N, agg_j, 0)

            def upd_j(j, _):
                va0 = _row(a0_v, j)
                vf0 = _row(f0_v, j)
                acc = _tree_sum(
                    [va0[c] * _row(wm_v, (4 * l + 0) * NF + c)
                     for c in range(NF)]
                    + [vf0[c] * _row(wm_v, (4 * l + 1) * NF + c)
                       for c in range(NF)])
                new1 = []
                for k in range(D):
                    q = k * N + j
                    va1 = _row(a1_v, q)
                    vf1 = _row(f1_v, q)
                    new1.append(_tree_sum(
                        [va1[c] * _row(wm_v, (4 * l + 2) * NF + c)
                         for c in range(NF)]
                        + [vf1[c] * _row(wm_v, (4 * l + 3) * NF + c)
                           for c in range(NF)]))
                _set_row(f0_v, j, jnp.maximum(acc, 0.0))
                for k in range(D):
                    _set_row(f1_v, k * N + j, new1[k])
                return 0

            lax.fori_loop(0, N, upd_j, 0)

        def out_body(n, _):
            row = _zero()
            for k in range(D):
                s = jnp.sum(_row(f1_v, k * N + n) * wout)
                pk = plsc.load_gather(x_v, [_bcast_i32((g * 8 + k) * N + n)])
                row = row + (s + pk) * eye[k]
            _set_row(out_v, g * N + n, row)
            return 0

        lax.fori_loop(0, N, out_body, 0)
        return 0

    lax.fori_loop(0, GPW, graph_body, 0)
    pltpu.sync_copy(out_v, out_hbm.at[pl.ds(base * N * NF, GPW * N * NF)])


def _pair_tables():
    i = np.repeat(np.arange(N, dtype=np.int32), N)
    j = np.tile(np.arange(N, dtype=np.int32), N)
    return np.stack([i, j])


@jax.jit
def kernel(pos, v, z, Wr, Wmix, w_embed, w_vinit, w_out):
    posr = pos.reshape(B, N, D).transpose(0, 2, 1)
    vr = v.reshape(B, N, D).transpose(0, 2, 1)
    zf = z.astype(_F32).reshape(B, 1, N)
    x = jnp.concatenate(
        [posr, vr, zf, jnp.zeros((B, 1, N), _F32)],
        axis=1).reshape(B * 8 * N)
    wr = Wr.reshape(N_LAYERS * 3 * N_RBF * NF)
    wm = Wmix.reshape(N_LAYERS * 4 * NF * NF)
    ws = jnp.concatenate([
        jnp.stack([w_embed, w_vinit, w_out,
                   jnp.asarray(np.linspace(0.0, 4.0, N_RBF,
                                           dtype=np.float32))]),
        jnp.asarray(np.eye(D, NF, dtype=np.float32)),
        jnp.zeros((1, NF), _F32),
    ]).reshape(8 * NF)
    idx = jnp.asarray(_pair_tables().reshape(-1))

    mesh = plsc.VectorSubcoreMesh(core_axis_name="c", subcore_axis_name="s",
                                  num_cores=2, num_subcores=16)
    out = pl.kernel(
        _sc_body,
        out_type=jax.ShapeDtypeStruct((B * N * NF,), _F32),
        mesh=mesh,
        compiler_params=pltpu.CompilerParams(needs_layout_passes=False),
        scratch_types=[
            pltpu.VMEM((GPW * 8 * N,), _F32),
            pltpu.VMEM((GPW * N * NF,), _F32),
            pltpu.VMEM((NP,), _F32),
            pltpu.VMEM((D * NP,), _F32),
            pltpu.VMEM((NP * N_RBF,), _F32),
            pltpu.VMEM((N * NF,), _F32),
            pltpu.VMEM((D * N * NF,), _F32),
            pltpu.VMEM((N * NF,), _F32),
            pltpu.VMEM((D * N * NF,), _F32),
            pltpu.VMEM((N_LAYERS * 3 * N_RBF * NF,), _F32),
            pltpu.VMEM((N_LAYERS * 4 * NF * NF,), _F32),
            pltpu.VMEM((8 * NF,), _F32),
            pltpu.VMEM((2 * NP,), _I32),
        ],
    )(x, wr, wm, ws, idx)
    return out.reshape(B * N, NF)[:, :D]


# speedup vs baseline: 1.4642x; 1.4642x over previous
import jax
import jax.numpy as jnp
import numpy as np
from jax import lax
from jax.experimental import pallas as pl
from jax.experimental.pallas import tpu as pltpu
from jax.experimental.pallas import tpu_sc as plsc

B = 1024
N = 20
D = 3
NF = 16
N_RBF = 16
N_LAYERS = 3
NP = N * N
NW = 32
GPW = B // NW

_F32 = jnp.float32
_I32 = jnp.int32


def _zero():
    return jnp.full((NF,), 0.0, dtype=_F32)


def _tree_sum(terms):
    while len(terms) > 1:
        terms = [terms[i] + terms[i + 1] for i in range(0, len(terms) - 1, 2)] \
            + ([terms[-1]] if len(terms) % 2 else [])
    return terms[0]


def _bcast_i32(s):
    return jnp.full((NF,), 0, dtype=_I32) + s


def _row(ref, r):
    return ref[pl.ds(r * NF, NF)]


def _set_row(ref, r, val):
    ref[pl.ds(r * NF, NF)] = val


def _sc_body(x_hbm, wr_hbm, wm_hbm, ws_hbm, idx_hbm, out_hbm,
             x_v, out_v, r_v, dh_v, rbf_v, f0_v, f1_v, a0_v, a1_v,
             wr_v, wm_v, ws_v, idx_v, w_v):
    wid = lax.axis_index("s") * 2 + lax.axis_index("c")
    base = wid * GPW
    pltpu.sync_copy(x_hbm.at[pl.ds(base * 8 * N, GPW * 8 * N)], x_v)
    pltpu.sync_copy(wr_hbm, wr_v)
    pltpu.sync_copy(wm_hbm, wm_v)
    pltpu.sync_copy(ws_hbm, ws_v)
    pltpu.sync_copy(idx_hbm, idx_v)
    wemb = _row(ws_v, 0)
    wvin = _row(ws_v, 1)
    wout = _row(ws_v, 2)
    centers = _row(ws_v, 3)
    eye = [_row(ws_v, 4 + k) for k in range(D)]

    def graph_body(g, _):
        def geom_body(q, _):
            sl = pl.ds(q * 16, 16)
            ii = idx_v[sl]
            jj = idx_v[pl.ds(NP + q * 16, 16)]
            d = []
            for k in range(D):
                cbase = _bcast_i32((g * 8 + k) * N)
                xi = plsc.load_gather(x_v, [cbase + ii])
                xj = plsc.load_gather(x_v, [cbase + jj])
                d.append(xj - xi)
            r2 = d[0] * d[0] + d[1] * d[1] + d[2] * d[2] + 1e-8
            yi = plsc.bitcast(r2, _I32)
            yi = jnp.full((NF,), 0x5F3759DF, dtype=_I32) \
                - lax.shift_right_logical(yi, 1)
            y = plsc.bitcast(yi, _F32)
            for _ in range(3):
                y = y * (1.5 - 0.5 * r2 * y * y)
            r = r2 * y
            r_v[sl] = jnp.where(ii == jj, jnp.full((NF,), 1e9, _F32), r)
            for k in range(D):
                dh_v[pl.ds(k * NP + q * 16, 16)] = d[k] * y
            return 0

        lax.fori_loop(0, NP // 16, geom_body, 0)

        def rbf_body(p, _):
            rb = plsc.load_gather(r_v, [_bcast_i32(p)])
            t = rb - centers
            _set_row(rbf_v, p, jnp.exp(-2.0 * t * t))
            return 0

        lax.fori_loop(0, NP, rbf_body, 0)

        def init_body(n, _):
            zf = plsc.load_gather(x_v, [_bcast_i32((g * 8 + 6) * N + n)])
            _set_row(f0_v, n, zf * wemb)
            for k in range(D):
                vk = plsc.load_gather(
                    x_v, [_bcast_i32((g * 8 + 3 + k) * N + n)])
                _set_row(f1_v, k * N + n, vk * wvin)
            return 0

        lax.fori_loop(0, N, init_body, 0)

        for l in range(N_LAYERS):
            wrows = [[_row(wr_v, (3 * l + kk) * NF + k) for k in range(N_RBF)]
                     for kk in range(3)]

            def wp_lo(lo, _):
                def wp_hi(hi, _):
                    p = lo * N + hi
                    rv = _row(rbf_v, p)
                    for kk in range(3):
                        wv = _tree_sum([rv[k] * wrows[kk][k]
                                        for k in range(N_RBF)])
                        _set_row(w_v, 3 * p + kk, wv)
                    return 0

                lax.fori_loop(lo, N, wp_hi, 0)
                return 0

            lax.fori_loop(0, N, wp_lo, 0)

            def agg_j(j, _):
                def agg_i(i, acc):
                    a0, ax, ay, az = acc
                    p = i * N + j
                    ps = jnp.minimum(i, j) * N + jnp.maximum(i, j)
                    w0 = _row(w_v, 3 * ps)
                    w1 = _row(w_v, 3 * ps + 1)
                    w2 = _row(w_v, 3 * ps + 2)
                    pb = _bcast_i32(p)
                    dhx = plsc.load_gather(dh_v, [pb])
                    dhy = plsc.load_gather(dh_v, [pb + NP])
                    dhz = plsc.load_gather(dh_v, [pb + 2 * NP])
                    a0 = a0 + w0 * _row(f0_v, i)
                    ax = ax + w1 * _row(f1_v, i) + dhx * w2
                    ay = ay + w1 * _row(f1_v, N + i) + dhy * w2
                    az = az + w1 * _row(f1_v, 2 * N + i) + dhz * w2
                    return a0, ax, ay, az

                a0, ax, ay, az = lax.fori_loop(
                    0, N, agg_i, (_zero(), _zero(), _zero(), _zero()))
                _set_row(a0_v, j, a0)
                _set_row(a1_v, j, ax)
                _set_row(a1_v, N + j, ay)
                _set_row(a1_v, 2 * N + j, az)
                return 0

            lax.fori_loop(0, N, agg_j, 0)

            def upd_j(j, _):
                va0 = _row(a0_v, j)
                vf0 = _row(f0_v, j)
                acc = _tree_sum(
                    [va0[c] * _row(wm_v, (4 * l + 0) * NF + c)
                     for c in range(NF)]
                    + [vf0[c] * _row(wm_v, (4 * l + 1) * NF + c)
                       for c in range(NF)])
                new1 = []
                for k in range(D):
                    q = k * N + j
                    va1 = _row(a1_v, q)
                    vf1 = _row(f1_v, q)
                    new1.append(_tree_sum(
                        [va1[c] * _row(wm_v, (4 * l + 2) * NF + c)
                         for c in range(NF)]
                        + [vf1[c] * _row(wm_v, (4 * l + 3) * NF + c)
                           for c in range(NF)]))
                _set_row(f0_v, j, jnp.maximum(acc, 0.0))
                for k in range(D):
                    _set_row(f1_v, k * N + j, new1[k])
                return 0

            lax.fori_loop(0, N, upd_j, 0)

        def out_body(n, _):
            row = _zero()
            for k in range(D):
                s = jnp.sum(_row(f1_v, k * N + n) * wout)
                pk = plsc.load_gather(x_v, [_bcast_i32((g * 8 + k) * N + n)])
                row = row + (s + pk) * eye[k]
            _set_row(out_v, g * N + n, row)
            return 0

        lax.fori_loop(0, N, out_body, 0)
        return 0

    lax.fori_loop(0, GPW, graph_body, 0)
    pltpu.sync_copy(out_v, out_hbm.at[pl.ds(base * N * NF, GPW * N * NF)])


def _pair_tables():
    i = np.repeat(np.arange(N, dtype=np.int32), N)
    j = np.tile(np.arange(N, dtype=np.int32), N)
    return np.stack([i, j])


@jax.jit
def kernel(pos, v, z, Wr, Wmix, w_embed, w_vinit, w_out):
    posr = pos.reshape(B, N, D).transpose(0, 2, 1)
    vr = v.reshape(B, N, D).transpose(0, 2, 1)
    zf = z.astype(_F32).reshape(B, 1, N)
    x = jnp.concatenate(
        [posr, vr, zf, jnp.zeros((B, 1, N), _F32)],
        axis=1).reshape(B * 8 * N)
    wr = Wr.reshape(N_LAYERS * 3 * N_RBF * NF)
    wm = Wmix.reshape(N_LAYERS * 4 * NF * NF)
    ws = jnp.concatenate([
        jnp.stack([w_embed, w_vinit, w_out,
                   jnp.asarray(np.linspace(0.0, 4.0, N_RBF,
                                           dtype=np.float32))]),
        jnp.asarray(np.eye(D, NF, dtype=np.float32)),
        jnp.zeros((1, NF), _F32),
    ]).reshape(8 * NF)
    idx = jnp.asarray(_pair_tables().reshape(-1))

    mesh = plsc.VectorSubcoreMesh(core_axis_name="c", subcore_axis_name="s",
                                  num_cores=2, num_subcores=16)
    out = pl.kernel(
        _sc_body,
        out_type=jax.ShapeDtypeStruct((B * N * NF,), _F32),
        mesh=mesh,
        compiler_params=pltpu.CompilerParams(needs_layout_passes=False),
        scratch_types=[
            pltpu.VMEM((GPW * 8 * N,), _F32),
            pltpu.VMEM((GPW * N * NF,), _F32),
            pltpu.VMEM((NP,), _F32),
            pltpu.VMEM((D * NP,), _F32),
            pltpu.VMEM((NP * N_RBF,), _F32),
            pltpu.VMEM((N * NF,), _F32),
            pltpu.VMEM((D * N * NF,), _F32),
            pltpu.VMEM((N * NF,), _F32),
            pltpu.VMEM((D * N * NF,), _F32),
            pltpu.VMEM((N_LAYERS * 3 * N_RBF * NF,), _F32),
            pltpu.VMEM((N_LAYERS * 4 * NF * NF,), _F32),
            pltpu.VMEM((8 * NF,), _F32),
            pltpu.VMEM((2 * NP,), _I32),
            pltpu.VMEM((NP * 3 * NF,), _F32),
        ],
    )(x, wr, wm, ws, idx)
    return out.reshape(B * N, NF)[:, :D]
